# baseline (speedup 1.0000x reference)
import functools

import jax
import jax.numpy as jnp
from jax import lax
from jax.experimental import pallas as pl
from jax.experimental.pallas import tpu as pltpu
from jax.experimental.pallas import tpu_sc as plsc

N = 6400000
G = 1024
NW = 32
PER_W = N // NW
C = 20000
VPC = C // 16
NCHUNK = PER_W // C
VB = 125
NBLK = VPC // VB

LANE = lambda: lax.iota(jnp.int32, 16)

_GDN = lax.GatherDimensionNumbers(
    offset_dims=(), collapsed_slice_dims=(0,), start_index_map=(0,))


def _gather16(v, idx):
    return lax.gather(v, idx[:, None], _GDN, (1,),
                      mode=lax.GatherScatterMode.PROMISE_IN_BOUNDS)


def _vsum(v):
    lane = LANE()
    for sh in (8, 4, 2, 1):
        v = v + _gather16(v, jnp.bitwise_xor(lane, sh))
    return v


def _flush(table, cur_g, accs, presummed_n=False):
    g = jnp.maximum(cur_g, 0)
    lane = LANE()
    m0 = lane == 0
    for m, acc in enumerate(accs):
        total = acc if (m == 0 and presummed_n) else _vsum(acc)
        plsc.addupdate_scatter(
            table, [jnp.full((16,), m * G + g, jnp.int32)], total, mask=m0)


def _zero_accs():
    z = jnp.zeros((16,), jnp.float32)
    return (z, z, z, z, z, z)


def _sc_body(x_hbm, y_hbm, g_hbm, out_hbm,
             gbuf0, xbuf0, ybuf0, gbuf1, xbuf1, ybuf1, table):
    wid = lax.axis_index("s") * 2 + lax.axis_index("c")
    wbase = wid * PER_W
    lane = LANE()

    def zrow(j, _):
        table[pl.ds(j * 16, 16)] = jnp.zeros((16,), jnp.float32)
        return 0
    lax.fori_loop(0, 6 * G // 16, zrow, 0)

    def slow_vec(gv, xv, yv, cur_g, accs):
        _flush(table, cur_g, accs)

        def lane_body(j, _):
            jv = jnp.full((16,), j, jnp.int32)
            gj = _gather16(gv, jv)
            xj = _gather16(xv, jv)
            yj = _gather16(yv, jv)
            mv = jnp.where(
                lane == 0, 1.0,
                jnp.where(lane == 1, xj,
                          jnp.where(lane == 2, yj,
                                    jnp.where(lane == 3, xj * yj,
                                              jnp.where(lane == 4, xj * xj,
                                                        yj * yj)))))
            plsc.addupdate_scatter(table, [lane * G + gj], mv, mask=lane < 6)
            return 0

        lax.fori_loop(0, 16, lane_body, 0)

    def vec_body(bufs, vbase, i, carry):
        gbuf, xbuf, ybuf = bufs
        cur_g = carry[0]
        accs = carry[1:]
        base = vbase + i * 16
        gv = gbuf[pl.ds(base, 16)]
        xv = xbuf[pl.ds(base, 16)]
        yv = ybuf[pl.ds(base, 16)]
        uniform = jnp.logical_and(gv[0] == cur_g, gv[15] == cur_g)

        lax.cond(uniform,
                 lambda: None,
                 lambda: slow_vec(gv, xv, yv, cur_g, accs))

        zero = jnp.zeros((16,), jnp.float32)
        an, ax, ay, axy, axx, ayy = accs
        new_accs = (
            jnp.where(uniform, an + 1.0, zero),
            jnp.where(uniform, ax + xv, zero),
            jnp.where(uniform, ay + yv, zero),
            jnp.where(uniform, axy + xv * yv, zero),
            jnp.where(uniform, axx + xv * xv, zero),
            jnp.where(uniform, ayy + yv * yv, zero),
        )
        return (gv[15],) + new_accs

    def seg_scan(bufs, sbase, nvec, split):
        gbuf, xbuf, ybuf = bufs
        g0 = gbuf[pl.ds(sbase, 16)][0]
        g1 = gbuf[pl.ds(sbase + (nvec - 1) * 16, 16)][15]

        def funi():
            z = jnp.zeros((16,), jnp.float32)
            half = nvec // 2

            @plsc.parallel_loop(0, half, carry=(z,) * 10, unroll=2)
            def accs10(i, a):
                b0 = sbase + i * 32
                x0 = xbuf[pl.ds(b0, 16)]
                y0 = ybuf[pl.ds(b0, 16)]
                x1 = xbuf[pl.ds(b0 + 16, 16)]
                y1 = ybuf[pl.ds(b0 + 16, 16)]
                return (a[0] + x0, a[1] + y0, a[2] + x0 * y0,
                        a[3] + x0 * x0, a[4] + y0 * y0,
                        a[5] + x1, a[6] + y1, a[7] + x1 * y1,
                        a[8] + x1 * x1, a[9] + y1 * y1)

            ax = accs10[0] + accs10[5]
            ay = accs10[1] + accs10[6]
            axy = accs10[2] + accs10[7]
            axx = accs10[3] + accs10[8]
            ayy = accs10[4] + accs10[9]
            if nvec % 2:
                bt = sbase + (nvec - 1) * 16
                xt = xbuf[pl.ds(bt, 16)]
                yt = ybuf[pl.ds(bt, 16)]
                ax = ax + xt
                ay = ay + yt
                axy = axy + xt * yt
                axx = axx + xt * xt
                ayy = ayy + yt * yt
            nv = jnp.full((16,), float(nvec * 16), jnp.float32)
            _flush(table, g0, (nv, ax, ay, axy, axx, ayy), presummed_n=True)

        def fnon():
            if split > 0:
                h = nvec // 2
                seg_scan(bufs, sbase, h, split - 1)
                seg_scan(bufs, sbase + h * 16, nvec - h, split - 1)
            else:
                init = (g0,) + _zero_accs()
                final = lax.fori_loop(
                    0, nvec, functools.partial(vec_body, bufs, sbase), init)
                _flush(table, final[0], final[1:])

        lax.cond(g0 == g1, funi, fnon)

    def block_body(bufs, b, _):
        seg_scan(bufs, b * VB * 16, VB, 2)
        return 0

    sets = ((gbuf0, xbuf0, ybuf0), (gbuf1, xbuf1, ybuf1))

    def _copies(k, bufs, sem):
        hb = wbase + k * C
        return [pltpu.make_async_copy(h.at[pl.ds(hb, C)], b, sem)
                for h, b in zip((g_hbm, x_hbm, y_hbm), bufs)]

    @functools.partial(pl.run_scoped, sem=pltpu.SemaphoreType.DMA(()))
    def _(sem):
        for c in _copies(0, sets[0], sem):
            c.start()
        for k in range(NCHUNK):
            cur = sets[k % 2]
            for c in _copies(k, cur, sem):
                c.wait()
            if k + 1 < NCHUNK:
                for c in _copies(k + 1, sets[(k + 1) % 2], sem):
                    c.start()
            lax.fori_loop(0, NBLK, functools.partial(block_body, cur), 0)

    pltpu.sync_copy(table, out_hbm.at[wid])


def _finalize_body(parts_ref, out_ref):
    p = parts_ref[...]
    s = jnp.sum(p, axis=0).reshape(6, 1024)
    n = s[0:1]
    sx = s[1:2]
    sy = s[2:3]
    sxy = s[3:4]
    sxx = s[4:5]
    syy = s[5:6]
    safe_n = jnp.where(n > 0, n, 1.0)
    cov = sxy - sx * sy / safe_n
    vx = sxx - sx * sx / safe_n
    vy = syy - sy * sy / safe_n
    denom = jnp.sqrt(jnp.maximum(vx, 0.0) * jnp.maximum(vy, 0.0))
    denom_safe = jnp.where(denom > 0, denom, 1.0)
    corr = jnp.where(denom > 0, cov / denom_safe, 0.0)
    val = -(jnp.sum(corr * n) / jnp.sum(n))
    out_ref[...] = jnp.full((1, 1), val, jnp.float32)


@jax.jit
def kernel(pred, exp, group):
    g32 = group.astype(jnp.int32)
    mesh = plsc.VectorSubcoreMesh(core_axis_name="c", subcore_axis_name="s")
    sc = pl.kernel(
        _sc_body,
        out_type=jax.ShapeDtypeStruct((NW, 6 * G), jnp.float32),
        mesh=mesh,
        compiler_params=pltpu.CompilerParams(needs_layout_passes=False),
        scratch_types=[
            pltpu.VMEM((C,), jnp.int32),
            pltpu.VMEM((C,), jnp.float32),
            pltpu.VMEM((C,), jnp.float32),
            pltpu.VMEM((C,), jnp.int32),
            pltpu.VMEM((C,), jnp.float32),
            pltpu.VMEM((C,), jnp.float32),
            pltpu.VMEM((6 * G,), jnp.float32),
        ],
    )
    parts = sc(exp, pred, g32)
    res = pl.pallas_call(
        _finalize_body,
        out_shape=jax.ShapeDtypeStruct((1, 1), jnp.float32),
    )(parts)
    return res[0, 0]


# speedup vs baseline: 1.0986x; 1.0986x over previous
import functools

import jax
import jax.numpy as jnp
from jax import lax
from jax.experimental import pallas as pl
from jax.experimental.pallas import tpu as pltpu
from jax.experimental.pallas import tpu_sc as plsc

N = 6400000
G = 1024
NW = 32
PER_W = N // NW
C = 20000
VPC = C // 16
NCHUNK = PER_W // C
VB = 50
NBLK = VPC // VB

LANE = lambda: lax.iota(jnp.int32, 16)

_GDN = lax.GatherDimensionNumbers(
    offset_dims=(), collapsed_slice_dims=(0,), start_index_map=(0,))


def _gather16(v, idx):
    return lax.gather(v, idx[:, None], _GDN, (1,),
                      mode=lax.GatherScatterMode.PROMISE_IN_BOUNDS)


def _vsum(v):
    lane = LANE()
    for sh in (8, 4, 2, 1):
        v = v + _gather16(v, jnp.bitwise_xor(lane, sh))
    return v


def _flush(table, cur_g, accs, presummed_n=False):
    g = jnp.maximum(cur_g, 0)
    lane = LANE()
    m0 = lane == 0
    for m, acc in enumerate(accs):
        total = acc if (m == 0 and presummed_n) else _vsum(acc)
        plsc.addupdate_scatter(
            table, [jnp.full((16,), m * G + g, jnp.int32)], total, mask=m0)


def _zero_accs():
    z = jnp.zeros((16,), jnp.float32)
    return (z, z, z, z, z, z)


def _sc_body(x_hbm, y_hbm, g_hbm, out_hbm,
             gbuf0, xbuf0, ybuf0, gbuf1, xbuf1, ybuf1, table):
    wid = lax.axis_index("s") * 2 + lax.axis_index("c")
    wbase = wid * PER_W
    lane = LANE()

    def zrow(j, _):
        table[pl.ds(j * 16, 16)] = jnp.zeros((16,), jnp.float32)
        return 0
    lax.fori_loop(0, 6 * G // 16, zrow, 0)

    def slow_vec(gv, xv, yv, cur_g, accs):
        _flush(table, cur_g, accs)

        def lane_body(j, _):
            jv = jnp.full((16,), j, jnp.int32)
            gj = _gather16(gv, jv)
            xj = _gather16(xv, jv)
            yj = _gather16(yv, jv)
            mv = jnp.where(
                lane == 0, 1.0,
                jnp.where(lane == 1, xj,
                          jnp.where(lane == 2, yj,
                                    jnp.where(lane == 3, xj * yj,
                                              jnp.where(lane == 4, xj * xj,
                                                        yj * yj)))))
            plsc.addupdate_scatter(table, [lane * G + gj], mv, mask=lane < 6)
            return 0

        lax.fori_loop(0, 16, lane_body, 0)

    def vec_body(bufs, vbase, i, carry):
        gbuf, xbuf, ybuf = bufs
        cur_g = carry[0]
        accs = carry[1:]
        base = vbase + i * 16
        gv = gbuf[pl.ds(base, 16)]
        xv = xbuf[pl.ds(base, 16)]
        yv = ybuf[pl.ds(base, 16)]
        uniform = jnp.logical_and(gv[0] == cur_g, gv[15] == cur_g)

        lax.cond(uniform,
                 lambda: None,
                 lambda: slow_vec(gv, xv, yv, cur_g, accs))

        zero = jnp.zeros((16,), jnp.float32)
        an, ax, ay, axy, axx, ayy = accs
        new_accs = (
            jnp.where(uniform, an + 1.0, zero),
            jnp.where(uniform, ax + xv, zero),
            jnp.where(uniform, ay + yv, zero),
            jnp.where(uniform, axy + xv * yv, zero),
            jnp.where(uniform, axx + xv * xv, zero),
            jnp.where(uniform, ayy + yv * yv, zero),
        )
        return (gv[15],) + new_accs

    def seg_scan(bufs, sbase, nvec, split):
        gbuf, xbuf, ybuf = bufs
        g0 = gbuf[pl.ds(sbase, 16)][0]
        g1 = gbuf[pl.ds(sbase + (nvec - 1) * 16, 16)][15]

        def funi():
            z = jnp.zeros((16,), jnp.float32)
            half = nvec // 2

            @plsc.parallel_loop(0, half, carry=(z,) * 10, unroll=2)
            def accs10(i, a):
                b0 = sbase + i * 32
                x0 = xbuf[pl.ds(b0, 16)]
                y0 = ybuf[pl.ds(b0, 16)]
                x1 = xbuf[pl.ds(b0 + 16, 16)]
                y1 = ybuf[pl.ds(b0 + 16, 16)]
                return (a[0] + x0, a[1] + y0, a[2] + x0 * y0,
                        a[3] + x0 * x0, a[4] + y0 * y0,
                        a[5] + x1, a[6] + y1, a[7] + x1 * y1,
                        a[8] + x1 * x1, a[9] + y1 * y1)

            ax = accs10[0] + accs10[5]
            ay = accs10[1] + accs10[6]
            axy = accs10[2] + accs10[7]
            axx = accs10[3] + accs10[8]
            ayy = accs10[4] + accs10[9]
            if nvec % 2:
                bt = sbase + (nvec - 1) * 16
                xt = xbuf[pl.ds(bt, 16)]
                yt = ybuf[pl.ds(bt, 16)]
                ax = ax + xt
                ay = ay + yt
                axy = axy + xt * yt
                axx = axx + xt * xt
                ayy = ayy + yt * yt
            nv = jnp.full((16,), float(nvec * 16), jnp.float32)
            _flush(table, g0, (nv, ax, ay, axy, axx, ayy), presummed_n=True)

        def fnon():
            if split > 0:
                h = nvec // 2
                seg_scan(bufs, sbase, h, split - 1)
                seg_scan(bufs, sbase + h * 16, nvec - h, split - 1)
            else:
                init = (g0,) + _zero_accs()
                final = lax.fori_loop(
                    0, nvec, functools.partial(vec_body, bufs, sbase), init)
                _flush(table, final[0], final[1:])

        lax.cond(g0 == g1, funi, fnon)

    def block_body(bufs, b, _):
        seg_scan(bufs, b * VB * 16, VB, 2)
        return 0

    sets = ((gbuf0, xbuf0, ybuf0), (gbuf1, xbuf1, ybuf1))

    def _copies(k, bufs, sem):
        hb = wbase + k * C
        return [pltpu.make_async_copy(h.at[pl.ds(hb, C)], b, sem)
                for h, b in zip((g_hbm, x_hbm, y_hbm), bufs)]

    @functools.partial(pl.run_scoped, sem=pltpu.SemaphoreType.DMA(()))
    def _(sem):
        for c in _copies(0, sets[0], sem):
            c.start()
        for k in range(NCHUNK):
            cur = sets[k % 2]
            for c in _copies(k, cur, sem):
                c.wait()
            if k + 1 < NCHUNK:
                for c in _copies(k + 1, sets[(k + 1) % 2], sem):
                    c.start()
            lax.fori_loop(0, NBLK, functools.partial(block_body, cur), 0)

    pltpu.sync_copy(table, out_hbm.at[wid])


def _finalize_body(parts_ref, out_ref):
    p = parts_ref[...]
    s = jnp.sum(p, axis=0).reshape(6, 1024)
    n = s[0:1]
    sx = s[1:2]
    sy = s[2:3]
    sxy = s[3:4]
    sxx = s[4:5]
    syy = s[5:6]
    safe_n = jnp.where(n > 0, n, 1.0)
    cov = sxy - sx * sy / safe_n
    vx = sxx - sx * sx / safe_n
    vy = syy - sy * sy / safe_n
    denom = jnp.sqrt(jnp.maximum(vx, 0.0) * jnp.maximum(vy, 0.0))
    denom_safe = jnp.where(denom > 0, denom, 1.0)
    corr = jnp.where(denom > 0, cov / denom_safe, 0.0)
    val = -(jnp.sum(corr * n) / jnp.sum(n))
    out_ref[...] = jnp.full((1, 1), val, jnp.float32)


@jax.jit
def kernel(pred, exp, group):
    g32 = group.astype(jnp.int32)
    mesh = plsc.VectorSubcoreMesh(core_axis_name="c", subcore_axis_name="s")
    sc = pl.kernel(
        _sc_body,
        out_type=jax.ShapeDtypeStruct((NW, 6 * G), jnp.float32),
        mesh=mesh,
        compiler_params=pltpu.CompilerParams(needs_layout_passes=False),
        scratch_types=[
            pltpu.VMEM((C,), jnp.int32),
            pltpu.VMEM((C,), jnp.float32),
            pltpu.VMEM((C,), jnp.float32),
            pltpu.VMEM((C,), jnp.int32),
            pltpu.VMEM((C,), jnp.float32),
            pltpu.VMEM((C,), jnp.float32),
            pltpu.VMEM((6 * G,), jnp.float32),
        ],
    )
    parts = sc(exp, pred, g32)
    res = pl.pallas_call(
        _finalize_body,
        out_shape=jax.ShapeDtypeStruct((1, 1), jnp.float32),
    )(parts)
    return res[0, 0]
